# baseline (speedup 1.0000x reference)
import functools

import jax
import jax.numpy as jnp
from jax import lax
from jax.experimental import pallas as pl
from jax.experimental.pallas import tpu as pltpu
from jax.experimental.pallas import tpu_sc as plsc

N = 10000
NP = 10240
E = 320000
D_IN = 128
H = 64
B = 256
NC = 2
NS = 16
NW = NC * NS
EPT = NP
NCHUNK = EPT // 128
ROWS_PER_TILE = NP // NS

def _worker_id():
    return lax.axis_index("s") * NC + lax.axis_index("c")


def _deg_body(dst_hbm, out_hbm, dst_v, acc_v):
    wid = _worker_id()

    def zero(i, _):
        acc_v[pl.ds(i * 16, 16)] = jnp.zeros((16,), jnp.float32)
        return 0

    lax.fori_loop(0, NP // 16, zero, 0)
    pltpu.sync_copy(dst_hbm.at[wid], dst_v)
    ones = jnp.ones((16,), jnp.float32)

    def body(j, _):
        row = j // 8
        col = (j % 8) * 16
        idx = dst_v[row, pl.ds(col, 16)]
        plsc.addupdate_scatter(acc_v, [idx], ones)
        return 0

    lax.fori_loop(0, EPT // 16, body, 0)
    pltpu.sync_copy(acc_v, out_hbm.at[wid])


GC = 256
NGCH = EPT // GC
NBUF = 2
NROUND = NGCH // NBUF


def _scatter_body(g_hbm, src_hbm, dst_hbm, out_hbm,
                  src_v, dst_v, zbuf, shared, *bufs_and_sems):
    bufs = bufs_and_sems[:NBUF]
    gsems = bufs_and_sems[NBUF:2 * NBUF]
    ssems = bufs_and_sems[2 * NBUF:3 * NBUF]
    cid = lax.axis_index("c")
    sid = lax.axis_index("s")
    wid = sid * NC + cid

    def zero(i, _):
        zbuf[i // 4, pl.ds((i % 4) * 16, 16)] = jnp.zeros((16,), jnp.float32)
        return 0

    lax.fori_loop(0, 128 * 4, zero, 0)
    for q in range(ROWS_PER_TILE // 128):
        pltpu.sync_copy(zbuf, shared.at[pl.ds(sid * ROWS_PER_TILE + q * 128, 128)])
    pltpu.sync_copy(src_hbm.at[wid], src_v)
    pltpu.sync_copy(dst_hbm.at[wid], dst_v)
    plsc.subcore_barrier()

    def gather_start(c, buf, sem):
        pltpu.async_copy(g_hbm.at[src_v.at[c]], buf, sem)

    def gather_wait(c, buf, sem):
        pltpu.make_async_copy(g_hbm.at[src_v.at[c]], buf, sem).wait()

    def scatter_start(c, buf, sem):
        pltpu.async_copy(buf.at[pl.ds(0, 128)],
                         shared.at[dst_v.at[2 * c]], sem, add=True)
        pltpu.async_copy(buf.at[pl.ds(128, 128)],
                         shared.at[dst_v.at[2 * c + 1]], sem, add=True)

    def scatter_wait(c, buf, sem):
        pltpu.make_async_copy(buf.at[pl.ds(0, 128)],
                              shared.at[dst_v.at[2 * c]], sem).wait()
        pltpu.make_async_copy(buf.at[pl.ds(128, 128)],
                              shared.at[dst_v.at[2 * c + 1]], sem).wait()

    for k in range(NBUF):
        gather_start(k, bufs[k], gsems[k])

    def roundfn(t, _):
        c0 = t * NBUF
        for k in range(NBUF):
            gather_wait(c0 + k, bufs[k], gsems[k])
            scatter_start(c0 + k, bufs[k], ssems[k])
        for k in range(NBUF):
            @pl.when(t < NROUND - 1)
            def _(k=k):
                scatter_wait(c0 + k, bufs[k], ssems[k])
                gather_start(c0 + NBUF + k, bufs[k], gsems[k])
        return 0

    lax.fori_loop(0, NROUND, roundfn, 0)
    for k in range(NBUF):
        scatter_wait((NROUND - 1) * NBUF + k, bufs[k], ssems[k])
    plsc.subcore_barrier()
    pltpu.sync_copy(shared.at[pl.ds(sid * ROWS_PER_TILE, ROWS_PER_TILE)],
                    out_hbm.at[cid, pl.ds(sid * ROWS_PER_TILE, ROWS_PER_TILE)])


_POOL_CHUNK = NP // NW


def _pool_body(h_hbm, bi_hbm, osum, omax, ocnt,
               h_v, bi_v, sum_v, max_v, cnt_v):
    wid = _worker_id()
    base = wid * _POOL_CHUNK
    pltpu.sync_copy(h_hbm.at[pl.ds(base, _POOL_CHUNK)], h_v)
    pltpu.sync_copy(bi_hbm.at[pl.ds(base, _POOL_CHUNK)],
                    bi_v.at[pl.ds(0, _POOL_CHUNK)])
    neg_inf = jnp.full((16,), -jnp.inf, jnp.float32)
    zeros = jnp.zeros((16,), jnp.float32)

    def init(i, _):
        sum_v[i // 4, pl.ds((i % 4) * 16, 16)] = zeros
        max_v[i // 4, pl.ds((i % 4) * 16, 16)] = neg_inf
        return 0

    lax.fori_loop(0, B * 4, init, 0)

    def initc(i, _):
        cnt_v[pl.ds(i * 16, 16)] = zeros
        return 0

    lax.fori_loop(0, B // 16, initc, 0)
    count = jnp.minimum(_POOL_CHUNK, N - base)
    ones = jnp.ones((16,), jnp.float32)

    def hist(i, _):
        idx = bi_v[pl.ds(i * 16, 16)]
        plsc.addupdate_scatter(cnt_v, [idx], ones)
        return 0

    lax.fori_loop(0, count // 16, hist, 0)

    def row(r, _):
        seg = bi_v[pl.ds(r, 16)][0]
        for f in range(H // 16):
            sl = pl.ds(f * 16, 16)
            v = h_v[r, sl]
            max_v[seg, sl] = jnp.maximum(max_v[seg, sl], v)
            sum_v[seg, sl] = sum_v[seg, sl] + v
        return 0

    lax.fori_loop(0, count, row, 0)
    pltpu.sync_copy(sum_v, osum.at[wid])
    pltpu.sync_copy(max_v, omax.at[wid])
    pltpu.sync_copy(cnt_v, ocnt.at[wid])


@functools.cache
def _sc_kernels():
    mesh = plsc.VectorSubcoreMesh(core_axis_name="c", subcore_axis_name="s",
                                  num_cores=NC, num_subcores=NS)
    deg = pl.kernel(
        _deg_body,
        out_type=jax.ShapeDtypeStruct((NW, NP), jnp.float32),
        mesh=mesh,
        scratch_types=[
            pltpu.VMEM((EPT // 128, 128), jnp.int32),
            pltpu.VMEM((NP,), jnp.float32),
        ],
        compiler_params=pltpu.CompilerParams(needs_layout_passes=False),
    )
    scat = pl.kernel(
        _scatter_body,
        out_type=jax.ShapeDtypeStruct((NC, NP, H), jnp.float32),
        mesh=mesh,
        scratch_types=(
            [
                pltpu.VMEM((NGCH, GC), jnp.int32),
                pltpu.VMEM((EPT // 128, 128), jnp.int32),
                pltpu.VMEM((128, H), jnp.float32),
                pltpu.VMEM_SHARED((NP, H), jnp.float32),
            ]
            + [pltpu.VMEM((GC, H), jnp.float32)] * NBUF
            + [pltpu.SemaphoreType.DMA] * (2 * NBUF)
        ),
        compiler_params=pltpu.CompilerParams(use_tc_tiling_on_sc=False),
    )
    pool = pl.kernel(
        _pool_body,
        out_type=(
            jax.ShapeDtypeStruct((NW, B, H), jnp.float32),
            jax.ShapeDtypeStruct((NW, B, H), jnp.float32),
            jax.ShapeDtypeStruct((NW, B), jnp.float32),
        ),
        mesh=mesh,
        scratch_types=[
            pltpu.VMEM((_POOL_CHUNK, H), jnp.float32),
            pltpu.VMEM((_POOL_CHUNK + 16,), jnp.int32),
            pltpu.VMEM((B, H), jnp.float32),
            pltpu.VMEM((B, H), jnp.float32),
            pltpu.VMEM((B,), jnp.float32),
        ],
        compiler_params=pltpu.CompilerParams(needs_layout_passes=False),
    )
    return deg, scat, pool


_ROWS_BLK = 1280
_GRID = NP // _ROWS_BLK


def _tc1_body(degp, x, w, g_out, dinv_out):
    deg = jnp.sum(degp[:, :], axis=0) + 1.0
    dinv = lax.rsqrt(deg)
    dinv_out[:, :] = dinv[:, None]
    h = jnp.dot(x[:, :], w[:, :], preferred_element_type=jnp.float32)
    g_out[:, :] = h * dinv[:, None]


def _tc1(deg_p, x_p, W1):
    return pl.pallas_call(
        _tc1_body,
        grid=(_GRID,),
        in_specs=[
            pl.BlockSpec((NW, _ROWS_BLK), lambda i: (0, i)),
            pl.BlockSpec((_ROWS_BLK, D_IN), lambda i: (i, 0)),
            pl.BlockSpec((D_IN, H), lambda i: (0, 0)),
        ],
        out_specs=[
            pl.BlockSpec((_ROWS_BLK, H), lambda i: (i, 0)),
            pl.BlockSpec((_ROWS_BLK, 1), lambda i: (i, 0)),
        ],
        out_shape=[
            jax.ShapeDtypeStruct((NP, H), jnp.float32),
            jax.ShapeDtypeStruct((NP, 1), jnp.float32),
        ],
    )(deg_p, x_p, W1)


def _tcmid_body(acc, g, dinv, w, b, g_out):
    s = acc[0] + acc[1] + g[:, :]
    a = jnp.maximum(s * dinv[:, :] + b[:, :], 0.0)
    h = jnp.dot(a, w[:, :], preferred_element_type=jnp.float32)
    g_out[:, :] = h * dinv[:, :]


def _tcmid(acc, g, dinv, w, b):
    return pl.pallas_call(
        _tcmid_body,
        grid=(_GRID,),
        in_specs=[
            pl.BlockSpec((NC, _ROWS_BLK, H), lambda i: (0, i, 0)),
            pl.BlockSpec((_ROWS_BLK, H), lambda i: (i, 0)),
            pl.BlockSpec((_ROWS_BLK, 1), lambda i: (i, 0)),
            pl.BlockSpec((H, H), lambda i: (0, 0)),
            pl.BlockSpec((1, H), lambda i: (0, 0)),
        ],
        out_specs=pl.BlockSpec((_ROWS_BLK, H), lambda i: (i, 0)),
        out_shape=jax.ShapeDtypeStruct((NP, H), jnp.float32),
    )(acc, g, dinv, w, b)


def _tclast_body(acc, g, dinv, b, h_out):
    s = acc[0] + acc[1] + g[:, :]
    h_out[:, :] = jnp.maximum(s * dinv[:, :] + b[:, :], 0.0)


def _tclast(acc, g, dinv, b):
    return pl.pallas_call(
        _tclast_body,
        grid=(_GRID,),
        in_specs=[
            pl.BlockSpec((NC, _ROWS_BLK, H), lambda i: (0, i, 0)),
            pl.BlockSpec((_ROWS_BLK, H), lambda i: (i, 0)),
            pl.BlockSpec((_ROWS_BLK, 1), lambda i: (i, 0)),
            pl.BlockSpec((1, H), lambda i: (0, 0)),
        ],
        out_specs=pl.BlockSpec((_ROWS_BLK, H), lambda i: (i, 0)),
        out_shape=jax.ShapeDtypeStruct((NP, H), jnp.float32),
    )(acc, g, dinv, b)


def _readout_body(sump, maxp, cntp, wout, bout, out_o, xp_o):
    s = jnp.zeros((B, H), jnp.float32)
    m = jnp.full((B, H), -jnp.inf, jnp.float32)
    for i in range(NW):
        s = s + sump[i]
        m = jnp.maximum(m, maxp[i])
    cnt = jnp.sum(cntp[:, :], axis=0)
    mean = s / jnp.maximum(cnt, 1.0)[:, None]
    xp = jnp.concatenate([mean, m], axis=1)
    xp_o[:, :] = xp
    out_o[:, :] = jnp.dot(xp, wout[:, :], preferred_element_type=jnp.float32) + bout[:, :]


def _readout(sump, maxp, cntp, W_out, b_out):
    return pl.pallas_call(
        _readout_body,
        out_shape=[
            jax.ShapeDtypeStruct((B, 1), jnp.float32),
            jax.ShapeDtypeStruct((B, 2 * H), jnp.float32),
        ],
    )(sump, maxp, cntp, W_out, b_out)


def kernel(x, edge_index, batch_index, W1, b1, W2, b2, W3, b3, W4, b4,
           W_out, b_out):
    pad_e = NW * EPT - E
    src_p = jnp.concatenate(
        [edge_index[0], jnp.zeros((pad_e,), jnp.int32)]).reshape(NW, NGCH, GC)
    dst_p = jnp.concatenate(
        [edge_index[1], jnp.full((pad_e,), N, jnp.int32)]).reshape(NW, EPT // 128, 128)
    x_p = jnp.pad(x, ((0, NP - N), (0, 0)))
    bi_p = jnp.pad(batch_index, (0, NP - N))

    deg_kernel, scatter_kernel, pool_kernel = _sc_kernels()
    deg_p = deg_kernel(dst_p)
    g, dinv = _tc1(deg_p, x_p, W1)
    acc = scatter_kernel(g, src_p, dst_p)
    g = _tcmid(acc, g, dinv, W2, b1.reshape(1, H))
    acc = scatter_kernel(g, src_p, dst_p)
    g = _tcmid(acc, g, dinv, W3, b2.reshape(1, H))
    acc = scatter_kernel(g, src_p, dst_p)
    g = _tcmid(acc, g, dinv, W4, b3.reshape(1, H))
    acc = scatter_kernel(g, src_p, dst_p)
    h = _tclast(acc, g, dinv, b4.reshape(1, H))
    sump, maxp, cntp = pool_kernel(h, bi_p)
    out, xp = _readout(sump, maxp, cntp, W_out, b_out.reshape(1, 1))
    return (out, xp)


# speedup vs baseline: 1.1130x; 1.1130x over previous
import functools

import jax
import jax.numpy as jnp
from jax import lax
from jax.experimental import pallas as pl
from jax.experimental.pallas import tpu as pltpu
from jax.experimental.pallas import tpu_sc as plsc

N = 10000
NP = 10240
E = 320000
D_IN = 128
H = 64
B = 256
NC = 2
NS = 16
NW = NC * NS
EPT = NP
NCHUNK = EPT // 128
ROWS_PER_TILE = NP // NS

def _worker_id():
    return lax.axis_index("s") * NC + lax.axis_index("c")


def _deg_body(dst_hbm, out_hbm, dst_v, acc_v):
    wid = _worker_id()

    def zero(i, _):
        acc_v[pl.ds(i * 16, 16)] = jnp.zeros((16,), jnp.float32)
        return 0

    lax.fori_loop(0, NP // 16, zero, 0)
    pltpu.sync_copy(dst_hbm.at[wid], dst_v)
    ones = jnp.ones((16,), jnp.float32)

    def body(j, _):
        row = j // 8
        col = (j % 8) * 16
        idx = dst_v[row, pl.ds(col, 16)]
        plsc.addupdate_scatter(acc_v, [idx], ones)
        return 0

    lax.fori_loop(0, (2 * 58 * 128) // 16, body, 0)
    pltpu.sync_copy(acc_v, out_hbm.at[wid])


GC = 256
K0 = 58
K1 = 22
NBUF = 2
TOTCH = NS * (K0 + K1)


def _scatter_body(g_hbm, src_hbm, dst_hbm, out_hbm,
                  src_v, dst_v, zbuf, shared, *bufs_and_sems):
    bufs = bufs_and_sems[:NBUF]
    gsems = bufs_and_sems[NBUF:2 * NBUF]
    ssems = bufs_and_sems[2 * NBUF:3 * NBUF]
    cid = lax.axis_index("c")
    sid = lax.axis_index("s")
    wid = sid * NC + cid

    def zero(i, _):
        zbuf[i // 4, pl.ds((i % 4) * 16, 16)] = jnp.zeros((16,), jnp.float32)
        return 0

    lax.fori_loop(0, 128 * 4, zero, 0)
    for q in range(ROWS_PER_TILE // 128):
        pltpu.sync_copy(zbuf, shared.at[pl.ds(sid * ROWS_PER_TILE + q * 128, 128)])
    pltpu.sync_copy(src_hbm.at[wid], src_v)
    pltpu.sync_copy(dst_hbm.at[wid], dst_v)
    plsc.subcore_barrier()

    def gather_start(c, buf, sem):
        pltpu.async_copy(g_hbm.at[src_v.at[c]], buf, sem)

    def gather_wait(c, buf, sem):
        pltpu.make_async_copy(g_hbm.at[src_v.at[c]], buf, sem).wait()

    def scatter_start(c, buf, sem):
        pltpu.async_copy(buf.at[pl.ds(0, 128)],
                         shared.at[dst_v.at[2 * c]], sem, add=True)
        pltpu.async_copy(buf.at[pl.ds(128, 128)],
                         shared.at[dst_v.at[2 * c + 1]], sem, add=True)

    def scatter_wait(c, buf, sem):
        pltpu.make_async_copy(buf.at[pl.ds(0, 128)],
                              shared.at[dst_v.at[2 * c]], sem).wait()
        pltpu.make_async_copy(buf.at[pl.ds(128, 128)],
                              shared.at[dst_v.at[2 * c + 1]], sem).wait()

    nrounds = jnp.where(cid == 0, K0 // NBUF, K1 // NBUF)
    for k in range(NBUF):
        gather_start(k, bufs[k], gsems[k])

    def roundfn(t, _):
        c0 = t * NBUF
        for k in range(NBUF):
            gather_wait(c0 + k, bufs[k], gsems[k])
            scatter_start(c0 + k, bufs[k], ssems[k])
        for k in range(NBUF):
            @pl.when(t < nrounds - 1)
            def _(k=k):
                scatter_wait(c0 + k, bufs[k], ssems[k])
                gather_start(c0 + NBUF + k, bufs[k], gsems[k])
        return 0

    lax.fori_loop(0, nrounds, roundfn, 0)
    for k in range(NBUF):
        scatter_wait((nrounds - 1) * NBUF + k, bufs[k], ssems[k])
    plsc.subcore_barrier()
    pltpu.sync_copy(shared.at[pl.ds(sid * ROWS_PER_TILE, ROWS_PER_TILE)],
                    out_hbm.at[cid, pl.ds(sid * ROWS_PER_TILE, ROWS_PER_TILE)])


_POOL_CHUNK = NP // NW


def _pool_body(h_hbm, bi_hbm, osum, omax, ocnt,
               h_v, bi_v, sum_v, max_v, cnt_v):
    wid = _worker_id()
    base = wid * _POOL_CHUNK
    pltpu.sync_copy(h_hbm.at[pl.ds(base, _POOL_CHUNK)], h_v)
    pltpu.sync_copy(bi_hbm.at[pl.ds(base, _POOL_CHUNK)],
                    bi_v.at[pl.ds(0, _POOL_CHUNK)])
    neg_inf = jnp.full((16,), -jnp.inf, jnp.float32)
    zeros = jnp.zeros((16,), jnp.float32)

    def init(i, _):
        sum_v[i // 4, pl.ds((i % 4) * 16, 16)] = zeros
        max_v[i // 4, pl.ds((i % 4) * 16, 16)] = neg_inf
        return 0

    lax.fori_loop(0, B * 4, init, 0)

    def initc(i, _):
        cnt_v[pl.ds(i * 16, 16)] = zeros
        return 0

    lax.fori_loop(0, B // 16, initc, 0)
    count = jnp.minimum(_POOL_CHUNK, N - base)
    ones = jnp.ones((16,), jnp.float32)

    def hist(i, _):
        idx = bi_v[pl.ds(i * 16, 16)]
        plsc.addupdate_scatter(cnt_v, [idx], ones)
        return 0

    lax.fori_loop(0, count // 16, hist, 0)

    def row(r, _):
        seg = bi_v[pl.ds(r, 16)][0]
        for f in range(H // 16):
            sl = pl.ds(f * 16, 16)
            v = h_v[r, sl]
            max_v[seg, sl] = jnp.maximum(max_v[seg, sl], v)
            sum_v[seg, sl] = sum_v[seg, sl] + v
        return 0

    lax.fori_loop(0, count, row, 0)
    pltpu.sync_copy(sum_v, osum.at[wid])
    pltpu.sync_copy(max_v, omax.at[wid])
    pltpu.sync_copy(cnt_v, ocnt.at[wid])


@functools.cache
def _sc_kernels():
    mesh = plsc.VectorSubcoreMesh(core_axis_name="c", subcore_axis_name="s",
                                  num_cores=NC, num_subcores=NS)
    deg = pl.kernel(
        _deg_body,
        out_type=jax.ShapeDtypeStruct((NW, NP), jnp.float32),
        mesh=mesh,
        scratch_types=[
            pltpu.VMEM((2 * K0, 128), jnp.int32),
            pltpu.VMEM((NP,), jnp.float32),
        ],
        compiler_params=pltpu.CompilerParams(needs_layout_passes=False),
    )
    scat = pl.kernel(
        _scatter_body,
        out_type=jax.ShapeDtypeStruct((NC, NP, H), jnp.float32),
        mesh=mesh,
        scratch_types=(
            [
                pltpu.VMEM((K0, GC), jnp.int32),
                pltpu.VMEM((2 * K0, 128), jnp.int32),
                pltpu.VMEM((128, H), jnp.float32),
                pltpu.VMEM_SHARED((NP, H), jnp.float32),
            ]
            + [pltpu.VMEM((GC, H), jnp.float32)] * NBUF
            + [pltpu.SemaphoreType.DMA] * (2 * NBUF)
        ),
        compiler_params=pltpu.CompilerParams(use_tc_tiling_on_sc=False),
    )
    pool = pl.kernel(
        _pool_body,
        out_type=(
            jax.ShapeDtypeStruct((NW, B, H), jnp.float32),
            jax.ShapeDtypeStruct((NW, B, H), jnp.float32),
            jax.ShapeDtypeStruct((NW, B), jnp.float32),
        ),
        mesh=mesh,
        scratch_types=[
            pltpu.VMEM((_POOL_CHUNK, H), jnp.float32),
            pltpu.VMEM((_POOL_CHUNK + 16,), jnp.int32),
            pltpu.VMEM((B, H), jnp.float32),
            pltpu.VMEM((B, H), jnp.float32),
            pltpu.VMEM((B,), jnp.float32),
        ],
        compiler_params=pltpu.CompilerParams(needs_layout_passes=False),
    )
    return deg, scat, pool


_ROWS_BLK = 1280
_GRID = NP // _ROWS_BLK


def _tc1_body(degp, x, w, g_out, dinv_out):
    deg = jnp.sum(degp[:, :], axis=0) + 1.0
    dinv = lax.rsqrt(deg)
    dinv_out[:, :] = dinv[:, None]
    h = jnp.dot(x[:, :], w[:, :], preferred_element_type=jnp.float32)
    g_out[:, :] = h * dinv[:, None]


def _tc1(deg_p, x_p, W1):
    return pl.pallas_call(
        _tc1_body,
        grid=(_GRID,),
        in_specs=[
            pl.BlockSpec((NW, _ROWS_BLK), lambda i: (0, i)),
            pl.BlockSpec((_ROWS_BLK, D_IN), lambda i: (i, 0)),
            pl.BlockSpec((D_IN, H), lambda i: (0, 0)),
        ],
        out_specs=[
            pl.BlockSpec((_ROWS_BLK, H), lambda i: (i, 0)),
            pl.BlockSpec((_ROWS_BLK, 1), lambda i: (i, 0)),
        ],
        out_shape=[
            jax.ShapeDtypeStruct((NP, H), jnp.float32),
            jax.ShapeDtypeStruct((NP, 1), jnp.float32),
        ],
    )(deg_p, x_p, W1)


def _tcmid_body(acc, g, dinv, w, b, g_out):
    s = acc[0] + acc[1] + g[:, :]
    a = jnp.maximum(s * dinv[:, :] + b[:, :], 0.0)
    h = jnp.dot(a, w[:, :], preferred_element_type=jnp.float32)
    g_out[:, :] = h * dinv[:, :]


def _tcmid(acc, g, dinv, w, b):
    return pl.pallas_call(
        _tcmid_body,
        grid=(_GRID,),
        in_specs=[
            pl.BlockSpec((NC, _ROWS_BLK, H), lambda i: (0, i, 0)),
            pl.BlockSpec((_ROWS_BLK, H), lambda i: (i, 0)),
            pl.BlockSpec((_ROWS_BLK, 1), lambda i: (i, 0)),
            pl.BlockSpec((H, H), lambda i: (0, 0)),
            pl.BlockSpec((1, H), lambda i: (0, 0)),
        ],
        out_specs=pl.BlockSpec((_ROWS_BLK, H), lambda i: (i, 0)),
        out_shape=jax.ShapeDtypeStruct((NP, H), jnp.float32),
    )(acc, g, dinv, w, b)


def _tclast_body(acc, g, dinv, b, h_out):
    s = acc[0] + acc[1] + g[:, :]
    h_out[:, :] = jnp.maximum(s * dinv[:, :] + b[:, :], 0.0)


def _tclast(acc, g, dinv, b):
    return pl.pallas_call(
        _tclast_body,
        grid=(_GRID,),
        in_specs=[
            pl.BlockSpec((NC, _ROWS_BLK, H), lambda i: (0, i, 0)),
            pl.BlockSpec((_ROWS_BLK, H), lambda i: (i, 0)),
            pl.BlockSpec((_ROWS_BLK, 1), lambda i: (i, 0)),
            pl.BlockSpec((1, H), lambda i: (0, 0)),
        ],
        out_specs=pl.BlockSpec((_ROWS_BLK, H), lambda i: (i, 0)),
        out_shape=jax.ShapeDtypeStruct((NP, H), jnp.float32),
    )(acc, g, dinv, b)


def _readout_body(sump, maxp, cntp, wout, bout, out_o, xp_o):
    s = jnp.zeros((B, H), jnp.float32)
    m = jnp.full((B, H), -jnp.inf, jnp.float32)
    for i in range(NW):
        s = s + sump[i]
        m = jnp.maximum(m, maxp[i])
    cnt = jnp.sum(cntp[:, :], axis=0)
    mean = s / jnp.maximum(cnt, 1.0)[:, None]
    xp = jnp.concatenate([mean, m], axis=1)
    xp_o[:, :] = xp
    out_o[:, :] = jnp.dot(xp, wout[:, :], preferred_element_type=jnp.float32) + bout[:, :]


def _readout(sump, maxp, cntp, W_out, b_out):
    return pl.pallas_call(
        _readout_body,
        out_shape=[
            jax.ShapeDtypeStruct((B, 1), jnp.float32),
            jax.ShapeDtypeStruct((B, 2 * H), jnp.float32),
        ],
    )(sump, maxp, cntp, W_out, b_out)


def kernel(x, edge_index, batch_index, W1, b1, W2, b2, W3, b3, W4, b4,
           W_out, b_out):
    pad_e = TOTCH * GC - E
    src_f = jnp.concatenate([edge_index[0], jnp.zeros((pad_e,), jnp.int32)])
    dst_f = jnp.concatenate([edge_index[1], jnp.full((pad_e,), N, jnp.int32)])
    src_c = src_f.reshape(TOTCH, GC)
    dst_c = dst_f.reshape(TOTCH, 2, 128)
    src_rows, dst_rows = [], []
    for sid in range(NS):
        base = sid * (K0 + K1)
        src_rows.append(src_c[base:base + K0])
        dst_rows.append(dst_c[base:base + K0])
        src_rows.append(jnp.pad(src_c[base + K0:base + K0 + K1],
                                ((0, K0 - K1), (0, 0)), constant_values=N))
        dst_rows.append(jnp.pad(dst_c[base + K0:base + K0 + K1],
                                ((0, K0 - K1), (0, 0), (0, 0)), constant_values=N))
    src_p = jnp.stack(src_rows)
    dst_p = jnp.stack(dst_rows).reshape(NW, 2 * K0, 128)
    x_p = jnp.pad(x, ((0, NP - N), (0, 0)))
    bi_p = jnp.pad(batch_index, (0, NP - N))

    deg_kernel, scatter_kernel, pool_kernel = _sc_kernels()
    deg_p = deg_kernel(dst_p)
    g, dinv = _tc1(deg_p, x_p, W1)
    acc = scatter_kernel(g, src_p, dst_p)
    g = _tcmid(acc, g, dinv, W2, b1.reshape(1, H))
    acc = scatter_kernel(g, src_p, dst_p)
    g = _tcmid(acc, g, dinv, W3, b2.reshape(1, H))
    acc = scatter_kernel(g, src_p, dst_p)
    g = _tcmid(acc, g, dinv, W4, b3.reshape(1, H))
    acc = scatter_kernel(g, src_p, dst_p)
    h = _tclast(acc, g, dinv, b4.reshape(1, H))
    sump, maxp, cntp = pool_kernel(h, bi_p)
    out, xp = _readout(sump, maxp, cntp, W_out, b_out.reshape(1, 1))
    return (out, xp)
